# baseline (speedup 1.0000x reference)
import jax
import jax.numpy as jnp
from jax import lax
from jax.experimental import pallas as pl
from jax.experimental.pallas import tpu as pltpu
from jax.experimental.pallas import tpu_sc as plsc

B = 64
GS = 2048
L = 16
CHUNK = GS // L
NW = 32
RPW = B // NW
NBINS = 128
NPASS = 5
HSIZE = NBINS * L
NVREG = NBINS // L
IDXB = 11
IMASK = (1 << IDXB) - 1
SH = (11, 18, 25, 11, 18)
TRANS = 2


def _desc_key(raw):
    bits = plsc.bitcast(raw, jnp.int32)
    m = jnp.where(bits >= 0, bits ^ jnp.int32(-2147483648), ~bits)
    return ~m


def _take16(x, idx):
    dn = lax.GatherDimensionNumbers(
        offset_dims=(), collapsed_slice_dims=(0,), start_index_map=(0,))
    return lax.gather(x, idx.reshape(16, 1), dn, (1,),
                      mode=lax.GatherScatterMode.PROMISE_IN_BOUNDS)


def _skew(q):
    qk = q & (CHUNK - 1)
    ql = lax.shift_right_logical(q, 7)
    return qk * L + ((ql + qk) & (L - 1))


def _sort_body(in_hbm, out_hbm, in_v, buf_a, buf_b, khigh, out_f, hist, offs):
    wid = lax.axis_index("s") * 2 + lax.axis_index("c")
    row0 = wid * RPW
    for rr in range(RPW):
        pltpu.sync_copy(in_hbm.at[row0 + rr], in_v.at[pl.ds(rr * GS, GS)])

    lane = lax.iota(jnp.int32, 16)
    lane_hist = lane * NBINS
    zeros16 = jnp.zeros((16,), jnp.int32)
    ones16 = jnp.ones((16,), jnp.int32)

    bufs = [buf_a, buf_b]

    def z_body(i, c):
        hist[pl.ds(i * L, L)] = zeros16
        return c
    lax.fori_loop(0, RPW * HSIZE // L, z_body, 0, unroll=4)

    @plsc.parallel_loop(0, CHUNK, unroll=4)
    def _b0(m):
        owner = lax.shift_right_logical(m, 3)
        for rr in range(RPW):
            q = m * L + lane
            raw = in_v[pl.ds(rr * GS + m * L, L)]
            key = _desc_key(raw)
            w = lax.shift_left(key, IDXB) | q
            plsc.store_scatter(buf_a, [_skew(q) + rr * GS], w)
            khigh[pl.ds(rr * GS + m * L, L)] = lax.shift_right_logical(key, 21)
            d = key & (NBINS - 1)
            plsc.addupdate_scatter(
                hist, [owner * NBINS + d + rr * HSIZE], ones16)

    for p in range(NPASS):
        last = p == NPASS - 1
        src = bufs[p % 2]
        dst = bufs[(p + 1) % 2]

        nv = 1 if last else NVREG

        def tot_body(l, T):
            out = []
            for rr in range(RPW):
                for j in range(nv):
                    h = hist[pl.ds(rr * HSIZE + l * NBINS + j * L, L)]
                    out.append(T[rr * nv + j] + h)
            return tuple(out)
        T = lax.fori_loop(0, L, tot_body, (zeros16,) * (RPW * nv),
                          unroll=2)

        R = []
        for rr in range(RPW):
            carry = jnp.int32(0)
            for j in range(nv):
                t = T[rr * nv + j]
                incl = plsc.cumsum(t)
                R.append((incl - t) + carry)
                carry = carry + jnp.sum(t)

        def run_body(l, Rc):
            out = []
            for rr in range(RPW):
                for j in range(nv):
                    addr = rr * HSIZE + l * NBINS + j * L
                    h = hist[pl.ds(addr, L)]
                    offs[pl.ds(addr, L)] = Rc[rr * nv + j]
                    if not last:
                        hist[pl.ds(addr, L)] = zeros16
                    out.append(Rc[rr * nv + j] + h)
            return tuple(out)
        lax.fori_loop(0, L, run_body, tuple(R), unroll=2)

        GB = 8

        def perm_body(g, c):
            for rr in range(RPW):
                curs, ds_, slots = [], [], []
                for i in range(GB):
                    k = g * GB + i
                    ridx = (lane + k) & (L - 1)
                    v = src[pl.ds(rr * GS + k * L, L)]
                    cur = _take16(v, ridx)
                    d = lax.shift_right_logical(cur, SH[p]) & (NBINS - 1)
                    curs.append(cur)
                    ds_.append(d)
                    slots.append(lane_hist + d + rr * HSIZE)
                raw = [plsc.load_gather(offs, [s]) for s in slots]
                offv = []
                for i in range(GB):
                    o = raw[i]
                    for j in range(i):
                        o = o + (ds_[j] == ds_[i]).astype(jnp.int32)
                    offv.append(o)
                for i in range(GB):
                    plsc.store_scatter(offs, [slots[i]], offv[i] + 1)
                for i in range(GB):
                    cur, off = curs[i], offv[i]
                    if p == TRANS:
                        idxv = cur & IMASK
                        kh = plsc.load_gather(khigh, [idxv + rr * GS])
                        scat = lax.shift_left(kh, IDXB) | idxv
                    elif last:
                        plsc.store_scatter(
                            out_f, [off + rr * GS],
                            (cur & IMASK).astype(jnp.float32))
                        continue
                    else:
                        scat = cur
                    plsc.store_scatter(dst, [_skew(off) + rr * GS], scat)
                    d2 = lax.shift_right_logical(scat, SH[p + 1]) & (NBINS - 1)
                    slot2 = (lax.shift_right_logical(off, 7) * NBINS
                             + d2 + rr * HSIZE)
                    plsc.addupdate_scatter(hist, [slot2], ones16)
            return c
        lax.fori_loop(0, CHUNK // GB, perm_body, 0)

    for rr in range(RPW):
        pltpu.sync_copy(out_f.at[pl.ds(rr * GS, GS)], out_hbm.at[row0 + rr])


def _sc_argsort(values):
    mesh = plsc.VectorSubcoreMesh(core_axis_name="c", subcore_axis_name="s")
    run = pl.kernel(
        _sort_body,
        out_type=jax.ShapeDtypeStruct((B, GS), jnp.float32),
        mesh=mesh,
        compiler_params=pltpu.CompilerParams(needs_layout_passes=False),
        scratch_types=[
            pltpu.VMEM((RPW * GS,), jnp.float32),
            pltpu.VMEM((RPW * GS,), jnp.int32),
            pltpu.VMEM((RPW * GS,), jnp.int32),
            pltpu.VMEM((RPW * GS,), jnp.int32),
            pltpu.VMEM((RPW * GS,), jnp.float32),
            pltpu.VMEM((RPW * HSIZE,), jnp.int32),
            pltpu.VMEM((RPW * HSIZE,), jnp.int32),
        ],
    )
    return run(values)


def kernel(x):
    values = x[0, :, :, -1]
    return _sc_argsort(values)


# speedup vs baseline: 1.0259x; 1.0259x over previous
import jax
import jax.numpy as jnp
from jax import lax
from jax.experimental import pallas as pl
from jax.experimental.pallas import tpu as pltpu
from jax.experimental.pallas import tpu_sc as plsc

B = 64
GS = 2048
L = 16
CHUNK = GS // L
NW = 32
RPW = B // NW
NBINS = 128
NPASS = 5
HSIZE = NBINS * L
NVREG = NBINS // L
IDXB = 11
IMASK = (1 << IDXB) - 1
SH = (11, 18, 25, 11, 18)
TRANS = 2


def _desc_key(raw):
    bits = plsc.bitcast(raw, jnp.int32)
    m = jnp.where(bits >= 0, bits ^ jnp.int32(-2147483648), ~bits)
    return ~m


def _take16(x, idx):
    dn = lax.GatherDimensionNumbers(
        offset_dims=(), collapsed_slice_dims=(0,), start_index_map=(0,))
    return lax.gather(x, idx.reshape(16, 1), dn, (1,),
                      mode=lax.GatherScatterMode.PROMISE_IN_BOUNDS)


def _skew(q):
    qk = q & (CHUNK - 1)
    ql = lax.shift_right_logical(q, 7)
    return qk * L + ((ql + qk) & (L - 1))


def _sort_body(in_hbm, out_hbm, in_v, buf_a, buf_b, khigh, out_f, hist, offs):
    wid = lax.axis_index("s") * 2 + lax.axis_index("c")
    row0 = wid * RPW
    for rr in range(RPW):
        pltpu.sync_copy(in_hbm.at[row0 + rr], in_v.at[pl.ds(rr * GS, GS)])

    lane = lax.iota(jnp.int32, 16)
    lane_hist = lane * NBINS
    zeros16 = jnp.zeros((16,), jnp.int32)
    ones16 = jnp.ones((16,), jnp.int32)

    bufs = [buf_a, buf_b]

    def z_body(i, c):
        hist[pl.ds(i * L, L)] = zeros16
        return c
    lax.fori_loop(0, RPW * HSIZE // L, z_body, 0, unroll=4)

    @plsc.parallel_loop(0, CHUNK, unroll=4)
    def _b0(m):
        owner = lax.shift_right_logical(m, 3)
        for rr in range(RPW):
            q = m * L + lane
            raw = in_v[pl.ds(rr * GS + m * L, L)]
            key = _desc_key(raw)
            w = lax.shift_left(key, IDXB) | q
            plsc.store_scatter(buf_a, [_skew(q) + rr * GS], w)
            khigh[pl.ds(rr * GS + m * L, L)] = lax.shift_right_logical(key, 21)
            d = key & (NBINS - 1)
            plsc.addupdate_scatter(
                hist, [owner * NBINS + d + rr * HSIZE], ones16)

    for p in range(NPASS):
        last = p == NPASS - 1
        src = bufs[p % 2]
        dst = bufs[(p + 1) % 2]

        nv = 1 if last else NVREG

        def tot_body(l, T):
            out = []
            for rr in range(RPW):
                for j in range(nv):
                    h = hist[pl.ds(rr * HSIZE + l * NBINS + j * L, L)]
                    out.append(T[rr * nv + j] + h)
            return tuple(out)
        T = lax.fori_loop(0, L, tot_body, (zeros16,) * (RPW * nv),
                          unroll=2)

        R = []
        for rr in range(RPW):
            carry = jnp.int32(0)
            for j in range(nv):
                t = T[rr * nv + j]
                incl = plsc.cumsum(t)
                R.append((incl - t) + carry)
                carry = carry + jnp.sum(t)

        def run_body(l, Rc):
            out = []
            for rr in range(RPW):
                for j in range(nv):
                    addr = rr * HSIZE + l * NBINS + j * L
                    h = hist[pl.ds(addr, L)]
                    offs[pl.ds(addr, L)] = Rc[rr * nv + j]
                    if not last:
                        hist[pl.ds(addr, L)] = zeros16
                    out.append(Rc[rr * nv + j] + h)
            return tuple(out)
        lax.fori_loop(0, L, run_body, tuple(R), unroll=2)

        GB = 4

        def perm_body(g, c):
            for rr in range(RPW):
                curs, ds_, slots = [], [], []
                for i in range(GB):
                    k = g * GB + i
                    ridx = (lane + k) & (L - 1)
                    v = src[pl.ds(rr * GS + k * L, L)]
                    cur = _take16(v, ridx)
                    d = lax.shift_right_logical(cur, SH[p]) & (NBINS - 1)
                    curs.append(cur)
                    ds_.append(d)
                    slots.append(lane_hist + d + rr * HSIZE)
                raw = [plsc.load_gather(offs, [s]) for s in slots]
                offv = []
                for i in range(GB):
                    o = raw[i]
                    for j in range(i):
                        o = o + (ds_[j] == ds_[i]).astype(jnp.int32)
                    offv.append(o)
                for i in range(GB):
                    plsc.store_scatter(offs, [slots[i]], offv[i] + 1)
                for i in range(GB):
                    cur, off = curs[i], offv[i]
                    if p == TRANS:
                        idxv = cur & IMASK
                        kh = plsc.load_gather(khigh, [idxv + rr * GS])
                        scat = lax.shift_left(kh, IDXB) | idxv
                    elif last:
                        plsc.store_scatter(
                            out_f, [off + rr * GS],
                            (cur & IMASK).astype(jnp.float32))
                        continue
                    else:
                        scat = cur
                    plsc.store_scatter(dst, [_skew(off) + rr * GS], scat)
                    d2 = lax.shift_right_logical(scat, SH[p + 1]) & (NBINS - 1)
                    slot2 = (lax.shift_right_logical(off, 7) * NBINS
                             + d2 + rr * HSIZE)
                    plsc.addupdate_scatter(hist, [slot2], ones16)
            return c
        lax.fori_loop(0, CHUNK // GB, perm_body, 0)

    for rr in range(RPW):
        pltpu.sync_copy(out_f.at[pl.ds(rr * GS, GS)], out_hbm.at[row0 + rr])


def _sc_argsort(values):
    mesh = plsc.VectorSubcoreMesh(core_axis_name="c", subcore_axis_name="s")
    run = pl.kernel(
        _sort_body,
        out_type=jax.ShapeDtypeStruct((B, GS), jnp.float32),
        mesh=mesh,
        compiler_params=pltpu.CompilerParams(needs_layout_passes=False),
        scratch_types=[
            pltpu.VMEM((RPW * GS,), jnp.float32),
            pltpu.VMEM((RPW * GS,), jnp.int32),
            pltpu.VMEM((RPW * GS,), jnp.int32),
            pltpu.VMEM((RPW * GS,), jnp.int32),
            pltpu.VMEM((RPW * GS,), jnp.float32),
            pltpu.VMEM((RPW * HSIZE,), jnp.int32),
            pltpu.VMEM((RPW * HSIZE,), jnp.int32),
        ],
    )
    return run(values)


def kernel(x):
    values = x[0, :, :, -1]
    return _sc_argsort(values)
